# baseline (speedup 1.0000x reference)
import functools

import jax
import jax.numpy as jnp
from jax import lax
from jax.experimental import pallas as pl
from jax.experimental.pallas import tpu as pltpu
from jax.experimental.pallas import tpu_sc as plsc

IN_SIZE = 1000000
OUT_SIZE = 64
BATCH = 16384

NC = 2
NS = 16
NW = NC * NS
B_PER_W = BATCH // NW
CHUNK = 128
N_CHUNKS = B_PER_W // CHUNK


def _gather_body(mat_hbm, idx_hbm, out_hbm, idx_v, rows_v, gsems, wsems):
    wid = lax.axis_index("s") * NC + lax.axis_index("c")
    base = wid * B_PER_W
    pltpu.sync_copy(idx_hbm.at[pl.ds(base, B_PER_W)], idx_v)
    gathers = []
    for j in range(N_CHUNKS):
        gathers.append(
            pltpu.async_copy(
                mat_hbm.at[idx_v.at[pl.ds(j * CHUNK, CHUNK)]],
                rows_v.at[pl.ds(j * CHUNK, CHUNK)],
                gsems.at[j],
            )
        )
    writes = []
    for j in range(N_CHUNKS):
        gathers[j].wait()
        writes.append(
            pltpu.async_copy(
                rows_v.at[pl.ds(j * CHUNK, CHUNK)],
                out_hbm.at[pl.ds(base + j * CHUNK, CHUNK)],
                wsems.at[j],
            )
        )
    for w in writes:
        w.wait()


@jax.jit
def _gather(x, mat):
    mesh = plsc.VectorSubcoreMesh(core_axis_name="c", subcore_axis_name="s")
    run = functools.partial(
        pl.kernel,
        out_type=jax.ShapeDtypeStruct((BATCH, OUT_SIZE), jnp.float32),
        mesh=mesh,
        scratch_types=[
            pltpu.VMEM((B_PER_W,), jnp.int32),
            pltpu.VMEM((B_PER_W, OUT_SIZE), jnp.float32),
            pltpu.SemaphoreType.DMA((N_CHUNKS,)),
            pltpu.SemaphoreType.DMA((N_CHUNKS,)),
        ],
        compiler_params=pltpu.CompilerParams(use_tc_tiling_on_sc=False),
    )(_gather_body)
    return run(mat, x)


def kernel(x, mat):
    return _gather(x, mat)


# speedup vs baseline: 3.7014x; 3.7014x over previous
import functools

import jax
import jax.numpy as jnp
from jax import lax
from jax.experimental import pallas as pl
from jax.experimental.pallas import tpu as pltpu
from jax.experimental.pallas import tpu_sc as plsc

IN_SIZE = 1000000
OUT_SIZE = 64
BATCH = 16384

NC = 2
NS = 16
NW = NC * NS
RANGE = IN_SIZE // NW
CHUNK_W = 512
N_CHUNKS = 62
TAIL_LO = (IN_SIZE // 128) * 128
MAX_OFF = TAIL_LO - CHUNK_W
SENTINEL = 0x7FFFFFF0
NSLOT = 16


def _body(mat3, idx_hbm, tail_hbm, out_hbm, idx_v, i_c, b_c, cbuf, stg,
          tail_v, gsems, wsem):
    wid = lax.axis_index("s") * NC + lax.axis_index("c")
    wlo = wid * RANGE
    whi = wlo + RANGE
    s0_base = pl.multiple_of((wlo // 128) * 128, 128)

    pltpu.sync_copy(idx_hbm, idx_v)
    pltpu.sync_copy(tail_hbm, tail_v)

    lanes = lax.iota(jnp.int32, 16)

    def chunk_off(c):
        vs = s0_base + c * CHUNK_W
        return pl.multiple_of(jnp.minimum(vs, MAX_OFF), 128)

    def start_chunk(c, b):
        off = chunk_off(c)
        for o in range(8):
            pltpu.async_copy(
                mat3.at[o, :, pl.ds(off, CHUNK_W)],
                cbuf.at[b, o],
                gsems.at[b],
            )

    def wait_chunk(b):
        for o in range(8):
            pltpu.make_async_copy(
                mat3.at[o, :, pl.ds(0, CHUNK_W)],
                cbuf.at[b, o],
                gsems.at[b],
            ).wait()

    def emit_row(wc, row_vecs, b_s):
        slot = wc & (NSLOT - 1)

        @pl.when(wc >= NSLOT)
        def _():
            pltpu.make_async_copy(
                stg.at[pl.ds(0, 1)], out_hbm.at[pl.ds(0, 1)], wsem
            ).wait()

        for k in range(4):
            stg[slot, pl.ds(k * 16, 16)] = row_vecs[k]
        pltpu.async_copy(
            stg.at[pl.ds(slot, 1)], out_hbm.at[pl.ds(b_s, 1)], wsem
        )
        return wc + 1

    def extract_main(wc, b, i_s, b_s, off):
        off2 = jnp.full((16,), i_s - off, dtype=jnp.int32)
        vecs = []
        for k in range(4):
            j = k * 16 + lanes
            vecs.append(
                plsc.load_gather(
                    cbuf.at[b],
                    [jax.lax.shift_right_logical(j, 3), j & 7, off2],
                )
            )
        return emit_row(wc, vecs, b_s)

    def extract_tail(wc, i_s, b_s):
        rr = jnp.full((16,), i_s - TAIL_LO, dtype=jnp.int32)
        vecs = [
            plsc.load_gather(tail_v, [rr, k * 16 + lanes]) for k in range(4)
        ]
        return emit_row(wc, vecs, b_s)

    def fill_body(v, _):
        i_c[pl.ds(v * 16, 16)] = jnp.full((16,), SENTINEL, dtype=jnp.int32)
        return 0

    lax.fori_loop(0, (BATCH + NSLOT) // 16 + 1, fill_body, 0)

    def scan_body(v, carry):
        cnt, wc = carry
        iv = idx_v[pl.ds(v * 16, 16)]
        bv = v * 16 + lanes
        mine = (iv >= wlo) & (iv < whi)
        m_main = mine & (iv < TAIL_LO)
        plsc.store_compressed(i_c.at[pl.ds(cnt, 16)], iv, mask=m_main)
        plsc.store_compressed(b_c.at[pl.ds(cnt, 16)], bv, mask=m_main)
        cnt = cnt + plsc.all_reduce_population_count(m_main)[0]

        m_tail = mine & (iv >= TAIL_LO)
        mt = jnp.where(m_tail, 1, 0)
        pc = plsc.all_reduce_population_count(m_tail)[0]

        def tail_cond(c):
            _, p, _ = c
            return p > 0

        def tail_step(c):
            m, p, wc_in = c
            e = plsc.all_reduce_ffs(m != 0)
            i_s = plsc.load_gather(idx_v, [v * 16 + e])[0]
            b_s = v * 16 + e[0]
            wc_out = extract_tail(wc_in, i_s, b_s)
            return (jnp.where(lanes == e, 0, m), p - 1, wc_out)

        _, _, wc = lax.while_loop(tail_cond, tail_step, (mt, pc, wc))
        return (cnt, wc)

    cnt, wc = lax.fori_loop(0, BATCH // 16, scan_body, (0, 0))
    nv = (cnt + 15) // 16

    start_chunk(0, 0)
    start_chunk(1, 1)

    def do_chunk(c, b, wc):
        wait_chunk(b)
        vs = s0_base + c * CHUNK_W
        off = chunk_off(c)

        def scan_matched(v, wc_in):
            iv = i_c[pl.ds(v * 16, 16)]
            m0 = (iv >= vs) & (iv < vs + CHUNK_W)
            mi = jnp.where(m0, 1, 0)
            pc = plsc.all_reduce_population_count(m0)[0]

            def w_cond(cr):
                _, p, _ = cr
                return p > 0

            def w_step(cr):
                m, p, wc2 = cr
                e = plsc.all_reduce_ffs(m != 0)
                i_s = plsc.load_gather(i_c, [v * 16 + e])[0]
                b_s = plsc.load_gather(b_c, [v * 16 + e])[0]
                wc3 = extract_main(wc2, b, i_s, b_s, off)
                return (jnp.where(lanes == e, 0, m), p - 1, wc3)

            _, _, wc_out = lax.while_loop(w_cond, w_step, (mi, pc, wc_in))
            return wc_out

        return lax.fori_loop(0, nv, scan_matched, wc)

    def outer(g, wc):
        wc = do_chunk(2 * g, 0, wc)

        @pl.when(2 * g + 2 < N_CHUNKS)
        def _():
            start_chunk(2 * g + 2, 0)

        wc = do_chunk(2 * g + 1, 1, wc)

        @pl.when(2 * g + 3 < N_CHUNKS)
        def _():
            start_chunk(2 * g + 3, 1)

        return wc

    wc = lax.fori_loop(0, N_CHUNKS // 2, outer, wc)

    def drain(_, x):
        pltpu.make_async_copy(
            stg.at[pl.ds(0, 1)], out_hbm.at[pl.ds(0, 1)], wsem
        ).wait()
        return x

    lax.fori_loop(0, jnp.minimum(wc, NSLOT), drain, 0)


@jax.jit
def _gather(x, mat):
    matT = jnp.transpose(mat)
    mat3 = jnp.reshape(matT, (8, 8, IN_SIZE))
    tail = lax.slice(mat, (TAIL_LO, 0), (IN_SIZE, OUT_SIZE))
    mesh = plsc.VectorSubcoreMesh(core_axis_name="c", subcore_axis_name="s")
    run = functools.partial(
        pl.kernel,
        out_type=jax.ShapeDtypeStruct((BATCH, OUT_SIZE), jnp.float32),
        mesh=mesh,
        scratch_types=[
            pltpu.VMEM((BATCH,), jnp.int32),
            pltpu.VMEM((BATCH + 32,), jnp.int32),
            pltpu.VMEM((BATCH + 32,), jnp.int32),
            pltpu.VMEM((2, 8, 8, CHUNK_W), jnp.float32),
            pltpu.VMEM((NSLOT, OUT_SIZE), jnp.float32),
            pltpu.VMEM((64, OUT_SIZE), jnp.float32),
            pltpu.SemaphoreType.DMA((2,)),
            pltpu.SemaphoreType.DMA,
        ],
        compiler_params=pltpu.CompilerParams(use_tc_tiling_on_sc=True, needs_layout_passes=False),
    )(_body)
    return run(mat3, x, tail)


def kernel(x, mat):
    return _gather(x, mat)


# speedup vs baseline: 4.1498x; 1.1212x over previous
import functools

import jax
import jax.numpy as jnp
from jax import lax
from jax.experimental import pallas as pl
from jax.experimental.pallas import tpu as pltpu
from jax.experimental.pallas import tpu_sc as plsc

IN_SIZE = 1000000
OUT_SIZE = 64
BATCH = 16384

NC = 2
NS = 16
NW = NC * NS
RANGE = IN_SIZE // NW
CHUNK_W = 512
N_CHUNKS = 62
TAIL_LO = (IN_SIZE // 128) * 128
MAX_OFF = TAIL_LO - CHUNK_W
SENTINEL = 0x7FFFFFF0
NSLOT = 16


def _body(mat3, idx_hbm, tail_hbm, out_hbm, idx_v, i_c, b_c, cbuf, stg,
          tail_v, gsems, wsem):
    wid = lax.axis_index("s") * NC + lax.axis_index("c")
    wlo = wid * RANGE
    whi = wlo + RANGE
    s0_base = pl.multiple_of((wlo // 128) * 128, 128)
    lanes = lax.iota(jnp.int32, 16)

    def chunk_off(c):
        vs = s0_base + c * CHUNK_W
        return pl.multiple_of(jnp.minimum(vs, MAX_OFF), 128)

    def start_chunk(c, b):
        pltpu.async_copy(
            mat3.at[:, :, pl.ds(chunk_off(c), CHUNK_W)], cbuf.at[b],
            gsems.at[b],
        )

    def wait_chunk(b):
        pltpu.make_async_copy(
            mat3.at[:, :, pl.ds(0, CHUNK_W)], cbuf.at[b], gsems.at[b]
        ).wait()

    start_chunk(0, 0)
    start_chunk(1, 1)

    pltpu.sync_copy(idx_hbm, idx_v)
    pltpu.sync_copy(tail_hbm, tail_v)

    def emit_row(wc, row_vecs, b_s):
        slot = wc & (NSLOT - 1)

        @pl.when(wc >= NSLOT)
        def _():
            pltpu.make_async_copy(
                stg.at[pl.ds(0, 1)], out_hbm.at[pl.ds(0, 1)], wsem
            ).wait()

        for k in range(4):
            stg[slot, pl.ds(k * 16, 16)] = row_vecs[k]
        pltpu.async_copy(
            stg.at[pl.ds(slot, 1)], out_hbm.at[pl.ds(b_s, 1)], wsem
        )
        return wc + 1

    def extract_main(wc, b, i_s, b_s, off):
        off2 = jnp.full((16,), i_s - off, dtype=jnp.int32)
        vecs = []
        for k in range(4):
            j = k * 16 + lanes
            vecs.append(
                plsc.load_gather(
                    cbuf.at[b],
                    [jax.lax.shift_right_logical(j, 3), j & 7, off2],
                )
            )
        return emit_row(wc, vecs, b_s)

    def extract_tail(wc, i_s, b_s):
        rr = jnp.full((16,), i_s - TAIL_LO, dtype=jnp.int32)
        vecs = [
            plsc.load_gather(tail_v, [rr, k * 16 + lanes]) for k in range(4)
        ]
        return emit_row(wc, vecs, b_s)

    def scan_body(v, cnt):
        iv = idx_v[pl.ds(v * 16, 16)]
        bv = v * 16 + lanes
        m = (iv >= wlo) & (iv < whi)
        plsc.store_compressed(i_c.at[pl.ds(cnt, 16)], iv, mask=m)
        plsc.store_compressed(b_c.at[pl.ds(cnt, 16)], bv, mask=m)
        return cnt + plsc.all_reduce_population_count(m)[0]

    cnt = lax.fori_loop(0, BATCH // 16, scan_body, 0)
    i_c[pl.ds(cnt, 16)] = jnp.full((16,), SENTINEL, dtype=jnp.int32)
    nv = (cnt + 15) // 16

    def do_chunk(c, b, wc):
        wait_chunk(b)
        vs = s0_base + c * CHUNK_W
        off = chunk_off(c)

        def scan_matched(extract):
            def scan_fn(v, wc_in):
                iv = i_c[pl.ds(v * 16, 16)]
                m0 = (iv >= vs) & (iv < vs + CHUNK_W)
                mi = jnp.where(m0, 1, 0)
                pc = plsc.all_reduce_population_count(m0)[0]

                def w_cond(cr):
                    _, p, _ = cr
                    return p > 0

                def w_step(cr):
                    m, p, wc2 = cr
                    e = plsc.all_reduce_ffs(m != 0)
                    i_s = plsc.load_gather(i_c, [v * 16 + e])[0]
                    b_s = plsc.load_gather(b_c, [v * 16 + e])[0]
                    wc3 = extract(wc2, i_s, b_s)
                    return (jnp.where(lanes == e, 0, m), p - 1, wc3)

                _, _, wc_out = lax.while_loop(w_cond, w_step, (mi, pc, wc_in))
                return wc_out

            return scan_fn

        main_fn = scan_matched(
            lambda wc2, i_s, b_s: extract_main(wc2, b, i_s, b_s, off)
        )
        tail_fn = scan_matched(extract_tail)
        return lax.cond(
            vs < TAIL_LO,
            lambda w: lax.fori_loop(0, nv, main_fn, w),
            lambda w: lax.fori_loop(0, nv, tail_fn, w),
            wc,
        )

    def outer(g, wc):
        wc = do_chunk(2 * g, 0, wc)

        @pl.when(2 * g + 2 < N_CHUNKS)
        def _():
            start_chunk(2 * g + 2, 0)

        wc = do_chunk(2 * g + 1, 1, wc)

        @pl.when(2 * g + 3 < N_CHUNKS)
        def _():
            start_chunk(2 * g + 3, 1)

        return wc

    wc = lax.fori_loop(0, N_CHUNKS // 2, outer, 0)

    def drain(_, x):
        pltpu.make_async_copy(
            stg.at[pl.ds(0, 1)], out_hbm.at[pl.ds(0, 1)], wsem
        ).wait()
        return x

    lax.fori_loop(0, jnp.minimum(wc, NSLOT), drain, 0)


@jax.jit
def _gather(x, mat):
    matT = jnp.transpose(mat)
    mat3 = jnp.reshape(matT, (8, 8, IN_SIZE))
    tail = lax.slice(mat, (TAIL_LO, 0), (IN_SIZE, OUT_SIZE))
    mesh = plsc.VectorSubcoreMesh(core_axis_name="c", subcore_axis_name="s")
    run = functools.partial(
        pl.kernel,
        out_type=jax.ShapeDtypeStruct((BATCH, OUT_SIZE), jnp.float32),
        mesh=mesh,
        scratch_types=[
            pltpu.VMEM((BATCH,), jnp.int32),
            pltpu.VMEM((BATCH + 32,), jnp.int32),
            pltpu.VMEM((BATCH + 32,), jnp.int32),
            pltpu.VMEM((2, 8, 8, CHUNK_W), jnp.float32),
            pltpu.VMEM((NSLOT, OUT_SIZE), jnp.float32),
            pltpu.VMEM((64, OUT_SIZE), jnp.float32),
            pltpu.SemaphoreType.DMA((2,)),
            pltpu.SemaphoreType.DMA,
        ],
        compiler_params=pltpu.CompilerParams(
            use_tc_tiling_on_sc=True, needs_layout_passes=False
        ),
    )(_body)
    return run(mat3, x, tail)


def kernel(x, mat):
    return _gather(x, mat)


# speedup vs baseline: 4.2139x; 1.0155x over previous
import functools

import jax
import jax.numpy as jnp
from jax import lax
from jax.experimental import pallas as pl
from jax.experimental.pallas import tpu as pltpu
from jax.experimental.pallas import tpu_sc as plsc

IN_SIZE = 1000000
OUT_SIZE = 64
BATCH = 16384

NC = 2
NS = 16
NW = NC * NS
RANGE = IN_SIZE // NW
CHUNK_W = 512
N_CHUNKS = 62
TAIL_LO = (IN_SIZE // 128) * 128
MAX_OFF = TAIL_LO - CHUNK_W
SENTINEL = 0x7FFFFFF0
NSLOT = 16


def _body(mat3, idx_hbm, tail_hbm, out_hbm, idx_v, p_c, cbuf, stg,
          tail_v, gsems, wsem):
    wid = lax.axis_index("s") * NC + lax.axis_index("c")
    wlo = wid * RANGE
    whi = wlo + RANGE
    s0_base = pl.multiple_of((wlo // 128) * 128, 128)
    lanes = lax.iota(jnp.int32, 16)

    def chunk_off(c):
        vs = s0_base + c * CHUNK_W
        return pl.multiple_of(jnp.minimum(vs, MAX_OFF), 128)

    def start_chunk(c, b):
        pltpu.async_copy(
            mat3.at[:, :, pl.ds(chunk_off(c), CHUNK_W)], cbuf.at[b],
            gsems.at[b],
        )

    def wait_chunk(b):
        pltpu.make_async_copy(
            mat3.at[:, :, pl.ds(0, CHUNK_W)], cbuf.at[b], gsems.at[b]
        ).wait()

    start_chunk(0, 0)
    start_chunk(1, 1)

    pltpu.sync_copy(idx_hbm, idx_v)
    pltpu.sync_copy(tail_hbm, tail_v)

    def emit_row(wc, row_vecs, b_s):
        slot = wc & (NSLOT - 1)

        @pl.when(wc >= NSLOT)
        def _():
            pltpu.make_async_copy(
                stg.at[pl.ds(0, 1)], out_hbm.at[pl.ds(0, 1)], wsem
            ).wait()

        for k in range(4):
            stg[slot, pl.ds(k * 16, 16)] = row_vecs[k]
        pltpu.async_copy(
            stg.at[pl.ds(slot, 1)], out_hbm.at[pl.ds(b_s, 1)], wsem
        )
        return wc + 1

    def extract_main(wc, b, i_s, b_s, off):
        off2 = jnp.full((16,), i_s - off, dtype=jnp.int32)
        vecs = []
        for k in range(4):
            j = k * 16 + lanes
            vecs.append(
                plsc.load_gather(
                    cbuf.at[b],
                    [jax.lax.shift_right_logical(j, 3), j & 7, off2],
                )
            )
        return emit_row(wc, vecs, b_s)

    def extract_tail(wc, i_s, b_s):
        rr = jnp.full((16,), i_s - TAIL_LO, dtype=jnp.int32)
        vecs = [
            plsc.load_gather(tail_v, [rr, k * 16 + lanes]) for k in range(4)
        ]
        return emit_row(wc, vecs, b_s)

    def scan_body(v, cnt):
        iv = idx_v[pl.ds(v * 16, 16)]
        bv = v * 16 + lanes
        m = (iv >= wlo) & (iv < whi)
        packed = jax.lax.shift_left(iv - s0_base, 14) | bv
        plsc.store_compressed(p_c.at[pl.ds(cnt, 16)], packed, mask=m)
        return cnt + plsc.all_reduce_population_count(m)[0]

    cnt = lax.fori_loop(0, BATCH // 16, scan_body, 0)
    p_c[pl.ds(cnt, 16)] = jnp.full((16,), SENTINEL, dtype=jnp.int32)
    nv = (cnt + 15) // 16

    def do_chunk(c, b, wc):
        wait_chunk(b)
        vs = s0_base + c * CHUNK_W
        off = chunk_off(c)

        plo = jax.lax.shift_left(vs - s0_base, 14)
        phi = jax.lax.shift_left(vs - s0_base + CHUNK_W, 14)

        def scan_matched(extract):
            def scan_fn(v, wc_in):
                pv = p_c[pl.ds(v * 16, 16)]
                m0 = (pv >= plo) & (pv < phi)
                mi = jnp.where(m0, 1, 0)
                pc = plsc.all_reduce_population_count(m0)[0]

                def w_cond(cr):
                    _, p, _ = cr
                    return p > 0

                def w_step(cr):
                    m, p, wc2 = cr
                    e = plsc.all_reduce_ffs(m != 0)
                    p_s = plsc.load_gather(p_c, [v * 16 + e])[0]
                    i_s = s0_base + jax.lax.shift_right_logical(p_s, 14)
                    b_s = p_s & (BATCH - 1)
                    wc3 = extract(wc2, i_s, b_s)
                    return (jnp.where(lanes == e, 0, m), p - 1, wc3)

                _, _, wc_out = lax.while_loop(w_cond, w_step, (mi, pc, wc_in))
                return wc_out

            return scan_fn

        main_fn = scan_matched(
            lambda wc2, i_s, b_s: extract_main(wc2, b, i_s, b_s, off)
        )
        tail_fn = scan_matched(extract_tail)
        return lax.cond(
            vs < TAIL_LO,
            lambda w: lax.fori_loop(0, nv, main_fn, w),
            lambda w: lax.fori_loop(0, nv, tail_fn, w),
            wc,
        )

    def outer(g, wc):
        wc = do_chunk(2 * g, 0, wc)

        @pl.when(2 * g + 2 < N_CHUNKS)
        def _():
            start_chunk(2 * g + 2, 0)

        wc = do_chunk(2 * g + 1, 1, wc)

        @pl.when(2 * g + 3 < N_CHUNKS)
        def _():
            start_chunk(2 * g + 3, 1)

        return wc

    wc = lax.fori_loop(0, N_CHUNKS // 2, outer, 0)

    def drain(_, x):
        pltpu.make_async_copy(
            stg.at[pl.ds(0, 1)], out_hbm.at[pl.ds(0, 1)], wsem
        ).wait()
        return x

    lax.fori_loop(0, jnp.minimum(wc, NSLOT), drain, 0)


@jax.jit
def _gather(x, mat):
    matT = jnp.transpose(mat)
    mat3 = jnp.reshape(matT, (8, 8, IN_SIZE))
    tail = lax.slice(mat, (TAIL_LO, 0), (IN_SIZE, OUT_SIZE))
    mesh = plsc.VectorSubcoreMesh(core_axis_name="c", subcore_axis_name="s")
    run = functools.partial(
        pl.kernel,
        out_type=jax.ShapeDtypeStruct((BATCH, OUT_SIZE), jnp.float32),
        mesh=mesh,
        scratch_types=[
            pltpu.VMEM((BATCH,), jnp.int32),
            pltpu.VMEM((BATCH + 32,), jnp.int32),
            pltpu.VMEM((2, 8, 8, CHUNK_W), jnp.float32),
            pltpu.VMEM((NSLOT, OUT_SIZE), jnp.float32),
            pltpu.VMEM((64, OUT_SIZE), jnp.float32),
            pltpu.SemaphoreType.DMA((2,)),
            pltpu.SemaphoreType.DMA,
        ],
        compiler_params=pltpu.CompilerParams(
            use_tc_tiling_on_sc=True, needs_layout_passes=False
        ),
    )(_body)
    return run(mat3, x, tail)


def kernel(x, mat):
    return _gather(x, mat)
